# baseline (speedup 1.0000x reference)
import functools

import jax
import jax.numpy as jnp
from jax import lax
from jax.experimental import pallas as pl
from jax.experimental.pallas import tpu as pltpu
from jax.experimental.pallas import tpu_sc as plsc

D = 128
TOTAL_ROWS = 200000
NW = 32
ROWS_W = 6256
ROWS_LAST = TOTAL_ROWS - (NW - 1) * ROWS_W
CHUNK = 256
NCHUNK = 25
GROUPS = CHUNK // 16


def _sc_matvec_body(x_hbm, a2_hbm, y_hbm, buf, a2_v, ybuf):
    cid = lax.axis_index("c")
    sid = lax.axis_index("s")
    wid = sid * 2 + cid
    start = wid * ROWS_W
    rows_w = jnp.where(wid == NW - 1, ROWS_LAST, ROWS_W)
    last_off = rows_w - CHUNK

    pltpu.sync_copy(a2_hbm, a2_v)

    flat_base = [(lax.iota(jnp.int32, 16) + (g * 16)) * D for g in range(GROUPS)]

    def chunk_body(i, _):
        off = jnp.minimum(i * CHUNK, last_off)
        row0 = start + off
        pltpu.sync_copy(x_hbm.at[pl.ds(row0 * D, CHUNK * D)], buf)

        def jbody(jb, accs):
            a2blk = a2_v[pl.ds(jb * 16, 16)]
            accs = list(accs)
            for dj in range(16):
                s = a2blk[dj]
                col = jnp.full((16,), jb * 16 + dj, jnp.int32)
                for g in range(GROUPS):
                    accs[g] = accs[g] + plsc.load_gather(
                        buf, [flat_base[g] + col]) * s
            return tuple(accs)

        zero = jnp.zeros((16,), jnp.float32)
        accs = lax.fori_loop(0, D // 16, jbody, (zero,) * GROUPS)
        for g in range(GROUPS):
            ybuf[pl.ds(g * 16, 16)] = accs[g]
        pltpu.sync_copy(ybuf, y_hbm.at[pl.ds(row0, CHUNK)])
        return 0

    lax.fori_loop(0, NCHUNK, chunk_body, 0)


@jax.jit
def _sc_matvec(x_flat, a2):
    mesh = plsc.VectorSubcoreMesh(core_axis_name="c", subcore_axis_name="s")
    return pl.kernel(
        _sc_matvec_body,
        out_type=jax.ShapeDtypeStruct((TOTAL_ROWS,), jnp.float32),
        mesh=mesh,
        compiler_params=pltpu.CompilerParams(needs_layout_passes=False),
        scratch_types=[
            pltpu.VMEM((CHUNK * D,), jnp.float32),
            pltpu.VMEM((D,), jnp.float32),
            pltpu.VMEM((CHUNK,), jnp.float32),
        ],
    )(x_flat, a2)


def _norm_body(y_ref, tgt_ref, a1_ref, o_ref):
    c = jnp.sum(tgt_ref[...] * a1_ref[...], axis=1, keepdims=True)
    e = y_ref[...] + c
    e = jnp.where(e > 0, e, 0.01 * e)
    o_ref[...] = e / jnp.sum(e, axis=1, keepdims=True)


@jax.jit
def _tc_normalize(y, tgt, a1):
    batch, n = y.shape
    return pl.pallas_call(
        _norm_body,
        out_shape=jax.ShapeDtypeStruct((batch, n), jnp.float32),
    )(y, tgt, a1)


def kernel(x, node_index, adj_mask, a):
    batch, node_num, d = x.shape
    tgt = jnp.take(x, node_index, axis=1)
    a1 = a[:d, 0].reshape(1, d)
    a2 = a[d:, 0]
    y = _sc_matvec(x.reshape(batch * node_num * d), a2)
    return _tc_normalize(y.reshape(batch, node_num), tgt, a1)


# speedup vs baseline: 4.8321x; 4.8321x over previous
import functools

import jax
import jax.numpy as jnp
from jax import lax
from jax.experimental import pallas as pl
from jax.experimental.pallas import tpu as pltpu
from jax.experimental.pallas import tpu_sc as plsc

D = 128
TOTAL_ROWS = 200000
NW = 32
ROWS_W = 6256
ROWS_LAST = TOTAL_ROWS - (NW - 1) * ROWS_W
CHUNK = 256
NCHUNK = 25
GROUPS = CHUNK // 16


def _sc_matvec_body(x_hbm, a2_hbm, y_hbm, buf, a2_v, ybuf):
    cid = lax.axis_index("c")
    sid = lax.axis_index("s")
    wid = sid * 2 + cid
    start = wid * ROWS_W
    rows_w = jnp.where(wid == NW - 1, ROWS_LAST, ROWS_W)
    last_off = rows_w - CHUNK

    pltpu.sync_copy(a2_hbm, a2_v.at[pl.ds(0, D)])
    pltpu.sync_copy(a2_hbm.at[pl.ds(0, 16)], a2_v.at[pl.ds(D, 16)])

    iota16 = lax.iota(jnp.int32, 16)
    row_base = [(iota16 + (g * 16)) * D for g in range(GROUPS)]

    def chunk_body(i, _):
        off = jnp.minimum(i * CHUNK, last_off)
        row0 = start + off
        pltpu.sync_copy(x_hbm.at[pl.ds(row0 * D, CHUNK * D)], buf)

        def jbody(j, accs):
            feat = (iota16 + j) & (D - 1)
            coeff = a2_v[pl.ds(j, 16)]
            return tuple(
                accs[g] + plsc.load_gather(buf, [row_base[g] + feat]) * coeff
                for g in range(GROUPS)
            )

        zero = jnp.zeros((16,), jnp.float32)
        accs = lax.fori_loop(0, D, jbody, (zero,) * GROUPS)
        for g in range(GROUPS):
            ybuf[pl.ds(g * 16, 16)] = accs[g]
        pltpu.sync_copy(ybuf, y_hbm.at[pl.ds(row0, CHUNK)])
        return 0

    lax.fori_loop(0, NCHUNK, chunk_body, 0)


@jax.jit
def _sc_matvec(x_flat, a2):
    mesh = plsc.VectorSubcoreMesh(core_axis_name="c", subcore_axis_name="s")
    return pl.kernel(
        _sc_matvec_body,
        out_type=jax.ShapeDtypeStruct((TOTAL_ROWS,), jnp.float32),
        mesh=mesh,
        compiler_params=pltpu.CompilerParams(needs_layout_passes=False),
        scratch_types=[
            pltpu.VMEM((CHUNK * D,), jnp.float32),
            pltpu.VMEM((D + 32,), jnp.float32),
            pltpu.VMEM((CHUNK,), jnp.float32),
        ],
    )(x_flat, a2)


def _norm_body(y_ref, tgt_ref, a1_ref, o_ref):
    c = jnp.sum(tgt_ref[...] * a1_ref[...], axis=1, keepdims=True)
    e = y_ref[...] + c
    e = jnp.where(e > 0, e, 0.01 * e)
    o_ref[...] = e / jnp.sum(e, axis=1, keepdims=True)


@jax.jit
def _tc_normalize(y, tgt, a1):
    batch, n = y.shape
    return pl.pallas_call(
        _norm_body,
        out_shape=jax.ShapeDtypeStruct((batch, n), jnp.float32),
    )(y, tgt, a1)


def kernel(x, node_index, adj_mask, a):
    batch, node_num, d = x.shape
    tgt = jnp.take(x, node_index, axis=1)
    a1 = a[:d, 0].reshape(1, d)
    a2 = a[d:, 0]
    y = _sc_matvec(x.reshape(batch * node_num * d), a2)
    return _tc_normalize(y.reshape(batch, node_num), tgt, a1)


# speedup vs baseline: 6.8507x; 1.4178x over previous
import functools

import jax
import jax.numpy as jnp
from jax import lax
from jax.experimental import pallas as pl
from jax.experimental.pallas import tpu as pltpu
from jax.experimental.pallas import tpu_sc as plsc

D = 128
TOTAL_ROWS = 200000
NW = 32
ROWS_W = 6256
ROWS_LAST = TOTAL_ROWS - (NW - 1) * ROWS_W
CHUNK = 256
NCHUNK = 25
GROUPS = CHUNK // 16


def _sc_matvec_body(x_hbm, a2_hbm, y_hbm, buf0, buf1, a2_v, ybuf0, ybuf1,
                    sem0, sem1, osem0, osem1):
    cid = lax.axis_index("c")
    sid = lax.axis_index("s")
    wid = sid * 2 + cid
    start = wid * ROWS_W
    rows_w = jnp.where(wid == NW - 1, ROWS_LAST, ROWS_W)
    last_off = rows_w - CHUNK

    pltpu.sync_copy(a2_hbm, a2_v.at[pl.ds(0, D)])
    pltpu.sync_copy(a2_hbm.at[pl.ds(0, 16)], a2_v.at[pl.ds(D, 16)])

    iota16 = lax.iota(jnp.int32, 16)
    row_base = [(iota16 + (g * 16)) * D for g in range(GROUPS)]

    bufs = [buf0, buf1]
    ybufs = [ybuf0, ybuf1]
    sems = [sem0, sem1]
    osems = [osem0, osem1]
    offs = [jnp.minimum(i * CHUNK, last_off) for i in range(NCHUNK)]

    def start_in(i):
        row0 = start + offs[i]
        return pltpu.async_copy(
            x_hbm.at[pl.ds(row0 * D, CHUNK * D)], bufs[i % 2], sems[i % 2])

    in_cp = {0: start_in(0)}
    out_cp = {}
    for i in range(NCHUNK):
        p = i % 2
        if i + 1 < NCHUNK:
            in_cp[i + 1] = start_in(i + 1)
        in_cp[i].wait()

        def jbody(j, accs):
            feat = (iota16 + j) & (D - 1)
            coeff = a2_v[pl.ds(j, 16)]
            return tuple(
                accs[g] + plsc.load_gather(bufs[p], [row_base[g] + feat])
                * coeff
                for g in range(GROUPS)
            )

        zero = jnp.zeros((16,), jnp.float32)
        accs = lax.fori_loop(0, D, jbody, (zero,) * GROUPS)
        if i - 2 in out_cp:
            out_cp[i - 2].wait()
        for g in range(GROUPS):
            ybufs[p][pl.ds(g * 16, 16)] = accs[g]
        out_cp[i] = pltpu.async_copy(
            ybufs[p], y_hbm.at[pl.ds(start + offs[i], CHUNK)], osems[p])
    out_cp[NCHUNK - 2].wait()
    out_cp[NCHUNK - 1].wait()


@jax.jit
def _sc_matvec(x_flat, a2):
    mesh = plsc.VectorSubcoreMesh(core_axis_name="c", subcore_axis_name="s")
    return pl.kernel(
        _sc_matvec_body,
        out_type=jax.ShapeDtypeStruct((TOTAL_ROWS,), jnp.float32),
        mesh=mesh,
        compiler_params=pltpu.CompilerParams(needs_layout_passes=False),
        scratch_types=[
            pltpu.VMEM((CHUNK * D,), jnp.float32),
            pltpu.VMEM((CHUNK * D,), jnp.float32),
            pltpu.VMEM((D + 32,), jnp.float32),
            pltpu.VMEM((CHUNK,), jnp.float32),
            pltpu.VMEM((CHUNK,), jnp.float32),
            pltpu.SemaphoreType.DMA,
            pltpu.SemaphoreType.DMA,
            pltpu.SemaphoreType.DMA,
            pltpu.SemaphoreType.DMA,
        ],
    )(x_flat, a2)


def _norm_body(y_ref, tgt_ref, a1_ref, o_ref):
    c = jnp.sum(tgt_ref[...] * a1_ref[...], axis=1, keepdims=True)
    e = y_ref[...] + c
    e = jnp.where(e > 0, e, 0.01 * e)
    o_ref[...] = e / jnp.sum(e, axis=1, keepdims=True)


@jax.jit
def _tc_normalize(y, tgt, a1):
    batch, n = y.shape
    return pl.pallas_call(
        _norm_body,
        out_shape=jax.ShapeDtypeStruct((batch, n), jnp.float32),
    )(y, tgt, a1)


def kernel(x, node_index, adj_mask, a):
    batch, node_num, d = x.shape
    tgt = jnp.take(x, node_index, axis=1)
    a1 = a[:d, 0].reshape(1, d)
    a2 = a[d:, 0]
    y = _sc_matvec(x.reshape(batch * node_num * d), a2)
    return _tc_normalize(y.reshape(batch, node_num), tgt, a1)
